# baseline (speedup 1.0000x reference)
import functools

import jax
import jax.numpy as jnp
from jax.experimental import pallas as pl
from jax.experimental.shard_map import shard_map
from jax.sharding import Mesh, PartitionSpec as P


def _proj_body(x_ref, w1_ref, s1_ref):
    s1 = jnp.dot(x_ref[...], w1_ref[...], preferred_element_type=jnp.float32)
    s1_ref[...] = s1.astype(jnp.bfloat16)


def _layer1_body(adj_ref, s1_ref, w2_ref, s2_ref):
    a = adj_ref[...].astype(jnp.bfloat16)
    h = jnp.dot(a, s1_ref[...], preferred_element_type=jnp.float32)
    x1 = jnp.where(h >= 0, h, 0.01 * h)
    s2 = jnp.dot(x1.astype(jnp.bfloat16), w2_ref[...],
                 preferred_element_type=jnp.float32)
    s2_ref[...] = s2.astype(jnp.bfloat16)


def _layer2_body(adj_ref, s2_ref, wout_ref, x3_ref, y_ref):
    a = adj_ref[...].astype(jnp.bfloat16)
    x3 = jnp.dot(a, s2_ref[...], preferred_element_type=jnp.float32)
    x3_ref[...] = x3
    logits = jnp.dot(x3.astype(jnp.bfloat16), wout_ref[...],
                     preferred_element_type=jnp.float32)
    y_ref[...] = jax.nn.sigmoid(logits)


def _shard_impl(x, adj, w1, w2_b, wout_b):
    rows, n = adj.shape
    nhid = w1.shape[1]
    nclass = wout_b.shape[1]
    br = 200
    grid = (rows // br,)

    s1 = pl.pallas_call(
        _proj_body,
        out_shape=jax.ShapeDtypeStruct((n, nhid), jnp.bfloat16),
    )(x, w1)

    s2_loc = pl.pallas_call(
        _layer1_body,
        grid=grid,
        in_specs=[
            pl.BlockSpec((br, n), lambda i: (i, 0)),
            pl.BlockSpec((n, nhid), lambda i: (0, 0)),
            pl.BlockSpec((nhid, nhid), lambda i: (0, 0)),
        ],
        out_specs=pl.BlockSpec((br, nhid), lambda i: (i, 0)),
        out_shape=jax.ShapeDtypeStruct((rows, nhid), jnp.bfloat16),
    )(adj, s1, w2_b)

    s2 = jax.lax.all_gather(s2_loc, "d", axis=0, tiled=True)

    x3_loc, y_loc = pl.pallas_call(
        _layer2_body,
        grid=grid,
        in_specs=[
            pl.BlockSpec((br, n), lambda i: (i, 0)),
            pl.BlockSpec((n, nhid), lambda i: (0, 0)),
            pl.BlockSpec((nhid, nclass), lambda i: (0, 0)),
        ],
        out_specs=[
            pl.BlockSpec((br, nhid), lambda i: (i, 0)),
            pl.BlockSpec((br, nclass), lambda i: (i, 0)),
        ],
        out_shape=[
            jax.ShapeDtypeStruct((rows, nhid), jnp.float32),
            jax.ShapeDtypeStruct((rows, nclass), jnp.float32),
        ],
    )(adj, s2, wout_b)

    return y_loc, x3_loc


def kernel(x, adj, W1, W2, W_out):
    n = adj.shape[0]
    devs = jax.devices()
    ndev = len(devs)
    while ndev > 1 and (n % ndev != 0 or (n // ndev) % 8 != 0):
        ndev -= 1
    mesh = Mesh(devs[:ndev], ("d",))
    w2_b = W2.astype(jnp.bfloat16)
    wout_b = W_out.astype(jnp.bfloat16)
    y, x3 = shard_map(
        _shard_impl,
        mesh=mesh,
        in_specs=(P(), P("d", None), P(), P(), P()),
        out_specs=(P("d", None), P("d", None)),
        check_rep=False,
    )(x, adj, W1, w2_b, wout_b)
    return (y, x3)


# speedup vs baseline: 2.8017x; 2.8017x over previous
import jax
import jax.numpy as jnp
from jax.experimental import pallas as pl
from jax.experimental import pallas as _pl
from jax.experimental.pallas import tpu as pltpu

_BR = 200


def _gcn_body(x_ref, adj_ref, w1_ref, w2_ref, wout_ref,
              x3_ref, y_ref, s1_ref, s2_ref):
    p = pl.program_id(0)
    i = pl.program_id(1)

    @pl.when(jnp.logical_and(p == 0, i == 0))
    def _():
        s1 = jnp.dot(x_ref[...], w1_ref[...],
                     preferred_element_type=jnp.float32)
        s1_ref[...] = s1.astype(jnp.bfloat16)

    a = adj_ref[...].astype(jnp.bfloat16)

    @pl.when(p == 0)
    def _():
        h = jnp.dot(a, s1_ref[...], preferred_element_type=jnp.float32)
        x1 = jnp.where(h >= 0, h, 0.01 * h)
        s2 = jnp.dot(x1.astype(jnp.bfloat16), w2_ref[...],
                     preferred_element_type=jnp.float32)
        s2_ref[pl.ds(i * _BR, _BR), :] = s2.astype(jnp.bfloat16)
        x3_ref[...] = jnp.zeros_like(x3_ref)
        y_ref[...] = jnp.zeros_like(y_ref)

    @pl.when(p == 1)
    def _():
        x3 = jnp.dot(a, s2_ref[...], preferred_element_type=jnp.float32)
        x3_ref[0] = x3
        logits = jnp.dot(x3.astype(jnp.bfloat16), wout_ref[...],
                         preferred_element_type=jnp.float32)
        y_ref[0] = jax.nn.sigmoid(logits)


def kernel(x, adj, W1, W2, W_out):
    n, nfeat = x.shape
    nhid = W1.shape[1]
    nclass = W_out.shape[1]
    r = n // _BR

    x3, y = pl.pallas_call(
        _gcn_body,
        grid=(2, r),
        in_specs=[
            pl.BlockSpec((n, nfeat), lambda p, i: (0, 0)),
            pl.BlockSpec((_BR, n), lambda p, i: (i, 0)),
            pl.BlockSpec((nfeat, nhid), lambda p, i: (0, 0)),
            pl.BlockSpec((nhid, nhid), lambda p, i: (0, 0)),
            pl.BlockSpec((nhid, nclass), lambda p, i: (0, 0)),
        ],
        out_specs=[
            pl.BlockSpec((1, _BR, nhid), lambda p, i: (p, i, 0)),
            pl.BlockSpec((1, _BR, nclass), lambda p, i: (p, i, 0)),
        ],
        out_shape=[
            jax.ShapeDtypeStruct((2, n, nhid), jnp.float32),
            jax.ShapeDtypeStruct((2, n, nclass), jnp.float32),
        ],
        scratch_shapes=[
            pltpu.VMEM((n, nhid), jnp.bfloat16),
            pltpu.VMEM((n, nhid), jnp.bfloat16),
        ],
    )(x, adj, W1, W2.astype(jnp.bfloat16), W_out.astype(jnp.bfloat16))

    return (y[1], x3[1])


# speedup vs baseline: 3.0627x; 1.0932x over previous
import jax
import jax.numpy as jnp
from jax.experimental import pallas as pl
from jax.experimental import pallas as _pl
from jax.experimental.pallas import tpu as pltpu

_BR = 400


def _gcn_body(x_ref, adj_ref, w1_ref, w2_ref, wout_ref,
              x3_ref, y_ref, s1_ref, s2_ref):
    p = pl.program_id(0)
    i = pl.program_id(1)

    @pl.when(jnp.logical_and(p == 0, i == 0))
    def _():
        s1 = jnp.dot(x_ref[...], w1_ref[...],
                     preferred_element_type=jnp.float32)
        s1_ref[...] = s1.astype(jnp.bfloat16)

    a = adj_ref[...].astype(jnp.bfloat16)

    @pl.when(p == 0)
    def _():
        h = jnp.dot(a, s1_ref[...], preferred_element_type=jnp.float32)
        x1 = jnp.where(h >= 0, h, 0.01 * h)
        s2 = jnp.dot(x1.astype(jnp.bfloat16), w2_ref[...],
                     preferred_element_type=jnp.float32)
        s2_ref[pl.ds(i * _BR, _BR), :] = s2.astype(jnp.bfloat16)
        x3_ref[...] = jnp.zeros_like(x3_ref)
        y_ref[...] = jnp.zeros_like(y_ref)

    @pl.when(p == 1)
    def _():
        x3 = jnp.dot(a, s2_ref[...], preferred_element_type=jnp.float32)
        x3_ref[0] = x3
        logits = jnp.dot(x3.astype(jnp.bfloat16), wout_ref[...],
                         preferred_element_type=jnp.float32)
        y_ref[0] = jax.nn.sigmoid(logits)


def kernel(x, adj, W1, W2, W_out):
    n, nfeat = x.shape
    nhid = W1.shape[1]
    nclass = W_out.shape[1]
    r = n // _BR

    x3, y = pl.pallas_call(
        _gcn_body,
        grid=(2, r),
        in_specs=[
            pl.BlockSpec((n, nfeat), lambda p, i: (0, 0)),
            pl.BlockSpec((_BR, n), lambda p, i: (i, 0)),
            pl.BlockSpec((nfeat, nhid), lambda p, i: (0, 0)),
            pl.BlockSpec((nhid, nhid), lambda p, i: (0, 0)),
            pl.BlockSpec((nhid, nclass), lambda p, i: (0, 0)),
        ],
        out_specs=[
            pl.BlockSpec((1, _BR, nhid), lambda p, i: (p, i, 0)),
            pl.BlockSpec((1, _BR, nclass), lambda p, i: (p, i, 0)),
        ],
        out_shape=[
            jax.ShapeDtypeStruct((2, n, nhid), jnp.float32),
            jax.ShapeDtypeStruct((2, n, nclass), jnp.float32),
        ],
        scratch_shapes=[
            pltpu.VMEM((n, nhid), jnp.bfloat16),
            pltpu.VMEM((n, nhid), jnp.bfloat16),
        ],
    )(x, adj, W1, W2.astype(jnp.bfloat16), W_out.astype(jnp.bfloat16))

    return (y[1], x3[1])
